# baseline (speedup 1.0000x reference)
import jax
import jax.numpy as jnp
from jax import lax
from jax.experimental import pallas as pl
from jax.experimental.pallas import tpu as pltpu
from jax.experimental.pallas import tpu_sc as plsc

N = 100000
EMBED = 128

_info = plsc.get_sparse_core_info()
NC, NS, L = _info.num_cores, _info.num_subcores, _info.num_lanes
NW = NC * NS

RPW = 3136
C = 112
NCHUNK = RPW // C
NBUF = 4


def _body(tids_hbm, nids_hbm, ttab_hbm, ntab_hbm, out_hbm, *scratch):
  tidx = scratch[0:NBUF]
  nidx = scratch[NBUF:2 * NBUF]
  tbuf = scratch[2 * NBUF:3 * NBUF]
  isem = scratch[3 * NBUF:4 * NBUF]
  gsem = scratch[4 * NBUF:5 * NBUF]
  asem = scratch[5 * NBUF:6 * NBUF]
  wsem = scratch[6 * NBUF:7 * NBUF]

  wid = lax.axis_index("s") * NC + lax.axis_index("c")
  base = jnp.minimum(wid * RPW, N - RPW)

  def issue_idx(chunk, b):
    row0 = base + chunk * C
    pltpu.async_copy(tids_hbm.at[pl.ds(row0, C)], tidx[b], isem[b])
    pltpu.async_copy(nids_hbm.at[pl.ds(row0, C)], nidx[b], isem[b])

  def wait_idx(b):
    pltpu.make_async_copy(tids_hbm.at[pl.ds(0, C)], tidx[b], isem[b]).wait()
    pltpu.make_async_copy(nids_hbm.at[pl.ds(0, C)], nidx[b], isem[b]).wait()

  def issue_tgather(b):
    pltpu.async_copy(ttab_hbm.at[tidx[b]], tbuf[b], gsem[b])

  def wait_tgather(b):
    pltpu.make_async_copy(ttab_hbm.at[pl.ds(0, C)], tbuf[b], gsem[b]).wait()

  def issue_nadd(b):
    pltpu.async_copy(ntab_hbm.at[nidx[b]], tbuf[b], asem[b], add=True)

  def wait_nadd(b):
    pltpu.make_async_copy(ntab_hbm.at[pl.ds(0, C)], tbuf[b], asem[b]).wait()

  def issue_write(chunk, b):
    row0 = base + chunk * C
    pltpu.async_copy(tbuf[b], out_hbm.at[pl.ds(row0, C)], wsem[b])

  def wait_write(b):
    pltpu.make_async_copy(tbuf[b], out_hbm.at[pl.ds(0, C)], wsem[b]).wait()

  for b in range(NBUF):
    issue_idx(b, b)
  for b in range(2):
    wait_idx(b)
    issue_tgather(b)
  wait_tgather(0)
  issue_nadd(0)

  def loop_body(k, carry):
    for s in range(NBUF):
      i = k * NBUF + s
      b = s
      b1 = (s + 1) % NBUF
      b2 = (s + 2) % NBUF

      wait_nadd(b)
      issue_write(i, b)

      @pl.when(i + 1 < NCHUNK)
      def _():
        wait_tgather(b1)
        issue_nadd(b1)

      @pl.when(i + 2 < NCHUNK)
      def _():
        @pl.when(i >= 2)
        def _():
          wait_write(b2)
        wait_idx(b2)
        issue_tgather(b2)

      @pl.when(i + NBUF < NCHUNK)
      def _():
        issue_idx(i + NBUF, b)
    return carry

  lax.fori_loop(0, NCHUNK // NBUF, loop_body, 0)

  for b in range(NBUF):
    wait_write(b)


_mesh = plsc.VectorSubcoreMesh(core_axis_name="c", subcore_axis_name="s")

_sc_embed = pl.kernel(
    _body,
    out_type=jax.ShapeDtypeStruct((N, EMBED), jnp.float32),
    mesh=_mesh,
    scratch_types=(
        [pltpu.VMEM((C,), jnp.int32) for _ in range(2 * NBUF)]
        + [pltpu.VMEM((C, EMBED), jnp.float32) for _ in range(NBUF)]
        + [pltpu.SemaphoreType.DMA for _ in range(4 * NBUF)]
    ),
)


@jax.jit
def kernel(token_ids, node_ids, token_table, node_table):
  return _sc_embed(token_ids.astype(jnp.int32), node_ids.astype(jnp.int32),
                   token_table, node_table)


# speedup vs baseline: 1.3892x; 1.3892x over previous
import jax
import jax.numpy as jnp
from jax import lax
from jax.experimental import pallas as pl
from jax.experimental.pallas import tpu as pltpu
from jax.experimental.pallas import tpu_sc as plsc

N = 100000
EMBED = 128

_info = plsc.get_sparse_core_info()
NC, NS, L = _info.num_cores, _info.num_subcores, _info.num_lanes
NW = NC * NS

RPW = 3136
C = 112
NCHUNK = RPW // C
NBUF = 4


def _body(tids_hbm, nids_hbm, ttab_hbm, ntab_hbm, out_hbm, *scratch):
  tidx = scratch[0:NBUF]
  nidx = scratch[NBUF:2 * NBUF]
  tbuf = scratch[2 * NBUF:3 * NBUF]
  isem = scratch[3 * NBUF:4 * NBUF]
  gsem = scratch[4 * NBUF:5 * NBUF]
  asem = scratch[5 * NBUF:6 * NBUF]
  wsem = scratch[6 * NBUF:7 * NBUF]
  nshared = scratch[7 * NBUF]

  sid = lax.axis_index("s")
  wid = sid * NC + lax.axis_index("c")
  base = jnp.minimum(wid * RPW, N - RPW)

  @pl.when(sid == 0)
  def _():
    pltpu.sync_copy(ntab_hbm, nshared)
  plsc.subcore_barrier()

  def issue_idx(chunk, b):
    row0 = base + chunk * C
    pltpu.async_copy(tids_hbm.at[pl.ds(row0, C)], tidx[b], isem[b])
    pltpu.async_copy(nids_hbm.at[pl.ds(row0, C)], nidx[b], isem[b])

  def wait_idx(b):
    pltpu.make_async_copy(tids_hbm.at[pl.ds(0, C)], tidx[b], isem[b]).wait()
    pltpu.make_async_copy(nids_hbm.at[pl.ds(0, C)], nidx[b], isem[b]).wait()

  def issue_tgather(b):
    pltpu.async_copy(ttab_hbm.at[tidx[b]], tbuf[b], gsem[b])

  def wait_tgather(b):
    pltpu.make_async_copy(ttab_hbm.at[pl.ds(0, C)], tbuf[b], gsem[b]).wait()

  def issue_nadd(b):
    pltpu.async_copy(nshared.at[nidx[b]], tbuf[b], asem[b], add=True)

  def wait_nadd(b):
    pltpu.make_async_copy(ntab_hbm.at[pl.ds(0, C)], tbuf[b], asem[b]).wait()

  def issue_write(chunk, b):
    row0 = base + chunk * C
    pltpu.async_copy(tbuf[b], out_hbm.at[pl.ds(row0, C)], wsem[b])

  def wait_write(b):
    pltpu.make_async_copy(tbuf[b], out_hbm.at[pl.ds(0, C)], wsem[b]).wait()

  for b in range(NBUF):
    issue_idx(b, b)
  for b in range(2):
    wait_idx(b)
    issue_tgather(b)
  wait_tgather(0)
  issue_nadd(0)

  def loop_body(k, carry):
    for s in range(NBUF):
      i = k * NBUF + s
      b = s
      b1 = (s + 1) % NBUF
      b2 = (s + 2) % NBUF

      wait_nadd(b)
      issue_write(i, b)

      @pl.when(i + 1 < NCHUNK)
      def _():
        wait_tgather(b1)
        issue_nadd(b1)

      @pl.when(i + 2 < NCHUNK)
      def _():
        @pl.when(i >= 2)
        def _():
          wait_write(b2)
        wait_idx(b2)
        issue_tgather(b2)

      @pl.when(i + NBUF < NCHUNK)
      def _():
        issue_idx(i + NBUF, b)
    return carry

  lax.fori_loop(0, NCHUNK // NBUF, loop_body, 0)

  for b in range(NBUF):
    wait_write(b)


_mesh = plsc.VectorSubcoreMesh(core_axis_name="c", subcore_axis_name="s")

_sc_embed = pl.kernel(
    _body,
    out_type=jax.ShapeDtypeStruct((N, EMBED), jnp.float32),
    mesh=_mesh,
    scratch_types=(
        [pltpu.VMEM((C,), jnp.int32) for _ in range(2 * NBUF)]
        + [pltpu.VMEM((C, EMBED), jnp.float32) for _ in range(NBUF)]
        + [pltpu.SemaphoreType.DMA for _ in range(4 * NBUF)]
        + [pltpu.VMEM_SHARED((1000, EMBED), jnp.float32)]
    ),
)


@jax.jit
def kernel(token_ids, node_ids, token_table, node_table):
  return _sc_embed(token_ids.astype(jnp.int32), node_ids.astype(jnp.int32),
                   token_table, node_table)


# speedup vs baseline: 1.5572x; 1.1209x over previous
import jax
import jax.numpy as jnp
from jax import lax
from jax.experimental import pallas as pl
from jax.experimental.pallas import tpu as pltpu
from jax.experimental.pallas import tpu_sc as plsc

N = 100000
EMBED = 128

_info = plsc.get_sparse_core_info()
NC, NS, L = _info.num_cores, _info.num_subcores, _info.num_lanes
NW = NC * NS

RPW = 3136
C = 112
NCHUNK = RPW // C
NBUF = 7


def _body(tids_hbm, nids_hbm, ttab_hbm, ntab_hbm, out_hbm, *scratch):
  tidx = scratch[0:NBUF]
  nidx = scratch[NBUF:2 * NBUF]
  tbuf = scratch[2 * NBUF:3 * NBUF]
  isem = scratch[3 * NBUF:4 * NBUF]
  gsem = scratch[4 * NBUF:5 * NBUF]
  asem = scratch[5 * NBUF:6 * NBUF]
  wsem = scratch[6 * NBUF:7 * NBUF]
  nshared = scratch[7 * NBUF]

  sid = lax.axis_index("s")
  wid = sid * NC + lax.axis_index("c")
  base = jnp.minimum(wid * RPW, N - RPW)

  @pl.when(sid == 0)
  def _():
    pltpu.sync_copy(ntab_hbm, nshared)
  plsc.subcore_barrier()

  def issue_idx(chunk, b):
    row0 = base + chunk * C
    pltpu.async_copy(tids_hbm.at[pl.ds(row0, C)], tidx[b], isem[b])
    pltpu.async_copy(nids_hbm.at[pl.ds(row0, C)], nidx[b], isem[b])

  def wait_idx(b):
    pltpu.make_async_copy(tids_hbm.at[pl.ds(0, C)], tidx[b], isem[b]).wait()
    pltpu.make_async_copy(nids_hbm.at[pl.ds(0, C)], nidx[b], isem[b]).wait()

  def issue_tgather(b):
    pltpu.async_copy(ttab_hbm.at[tidx[b]], tbuf[b], gsem[b])

  def wait_tgather(b):
    pltpu.make_async_copy(ttab_hbm.at[pl.ds(0, C)], tbuf[b], gsem[b]).wait()

  def issue_nadd(b):
    pltpu.async_copy(nshared.at[nidx[b]], tbuf[b], asem[b], add=True)

  def wait_nadd(b):
    pltpu.make_async_copy(ntab_hbm.at[pl.ds(0, C)], tbuf[b], asem[b]).wait()

  def issue_write(chunk, b):
    row0 = base + chunk * C
    pltpu.async_copy(tbuf[b], out_hbm.at[pl.ds(row0, C)], wsem[b])

  def wait_write(b):
    pltpu.make_async_copy(tbuf[b], out_hbm.at[pl.ds(0, C)], wsem[b]).wait()

  for b in range(NBUF):
    issue_idx(b, b)
  for b in range(4):
    wait_idx(b)
    issue_tgather(b)
  for b in range(2):
    wait_tgather(b)
    issue_nadd(b)

  def loop_body(k, carry):
    for s in range(NBUF):
      i = k * NBUF + s
      b = s
      b2 = (s + 2) % NBUF
      b4 = (s + 4) % NBUF

      wait_nadd(b)
      issue_write(i, b)

      @pl.when(i + 2 < NCHUNK)
      def _():
        wait_tgather(b2)
        issue_nadd(b2)

      @pl.when(i + 4 < NCHUNK)
      def _():
        @pl.when(i >= 3)
        def _():
          wait_write(b4)
        wait_idx(b4)
        issue_tgather(b4)

      @pl.when(i + NBUF < NCHUNK)
      def _():
        issue_idx(i + NBUF, b)
    return carry

  lax.fori_loop(0, NCHUNK // NBUF, loop_body, 0)

  for b in range(NBUF):
    wait_write(b)


_mesh = plsc.VectorSubcoreMesh(core_axis_name="c", subcore_axis_name="s")

_sc_embed = pl.kernel(
    _body,
    out_type=jax.ShapeDtypeStruct((N, EMBED), jnp.float32),
    mesh=_mesh,
    scratch_types=(
        [pltpu.VMEM((C,), jnp.int32) for _ in range(2 * NBUF)]
        + [pltpu.VMEM((C, EMBED), jnp.float32) for _ in range(NBUF)]
        + [pltpu.SemaphoreType.DMA for _ in range(4 * NBUF)]
        + [pltpu.VMEM_SHARED((1000, EMBED), jnp.float32)]
    ),
)


@jax.jit
def kernel(token_ids, node_ids, token_table, node_table):
  return _sc_embed(token_ids.astype(jnp.int32), node_ids.astype(jnp.int32),
                   token_table, node_table)


# speedup vs baseline: 1.5816x; 1.0157x over previous
import jax
import jax.numpy as jnp
from jax import lax
from jax.experimental import pallas as pl
from jax.experimental.pallas import tpu as pltpu
from jax.experimental.pallas import tpu_sc as plsc

N = 100000
EMBED = 128

_info = plsc.get_sparse_core_info()
NC, NS, L = _info.num_cores, _info.num_subcores, _info.num_lanes
NW = NC * NS

RPW = 3136
C = 112
NCHUNK = RPW // C
NBUF = 7


def _body(tids_hbm, nids_hbm, ttab_hbm, ntab_hbm, out_hbm, *scratch):
  tidx = scratch[0:NBUF]
  nidx = scratch[NBUF:2 * NBUF]
  tbuf = scratch[2 * NBUF:3 * NBUF]
  isem = scratch[3 * NBUF:4 * NBUF]
  gsem = scratch[4 * NBUF:5 * NBUF]
  asem = scratch[5 * NBUF:6 * NBUF]
  wsem = scratch[6 * NBUF:7 * NBUF]
  nshared = scratch[7 * NBUF]

  sid = lax.axis_index("s")
  wid = sid * NC + lax.axis_index("c")
  base = jnp.minimum(wid * RPW, N - RPW)

  @pl.when(sid == 0)
  def _():
    pltpu.sync_copy(ntab_hbm, nshared)
  plsc.subcore_barrier()

  def issue_idx(chunk, b):
    row0 = base + chunk * C
    pltpu.async_copy(tids_hbm.at[pl.ds(row0, C)], tidx[b], isem[b])
    pltpu.async_copy(nids_hbm.at[pl.ds(row0, C)], nidx[b], isem[b])

  def wait_idx(b):
    pltpu.make_async_copy(tids_hbm.at[pl.ds(0, C)], tidx[b], isem[b]).wait()
    pltpu.make_async_copy(nids_hbm.at[pl.ds(0, C)], nidx[b], isem[b]).wait()

  def issue_tgather(b):
    pltpu.async_copy(ttab_hbm.at[tidx[b]], tbuf[b], gsem[b])

  def wait_tgather(b):
    pltpu.make_async_copy(ttab_hbm.at[pl.ds(0, C)], tbuf[b], gsem[b]).wait()

  def issue_nadd(b):
    pltpu.async_copy(nshared.at[nidx[b]], tbuf[b], asem[b], add=True)

  def wait_nadd(b):
    pltpu.make_async_copy(ntab_hbm.at[pl.ds(0, C)], tbuf[b], asem[b]).wait()

  def issue_write(chunk, b):
    row0 = base + chunk * C
    pltpu.async_copy(tbuf[b], out_hbm.at[pl.ds(row0, C)], wsem[b])

  def wait_write(b):
    pltpu.make_async_copy(tbuf[b], out_hbm.at[pl.ds(0, C)], wsem[b]).wait()

  for b in range(NBUF):
    issue_idx(b, b)
  for b in range(5):
    wait_idx(b)
    issue_tgather(b)
  for b in range(3):
    wait_tgather(b)
    issue_nadd(b)

  def loop_body(k, carry):
    for s in range(NBUF):
      i = k * NBUF + s
      b = s
      b3 = (s + 3) % NBUF
      b5 = (s + 5) % NBUF

      wait_nadd(b)
      issue_write(i, b)

      @pl.when(i + 3 < NCHUNK)
      def _():
        wait_tgather(b3)
        issue_nadd(b3)

      @pl.when(i + 5 < NCHUNK)
      def _():
        @pl.when(i >= 2)
        def _():
          wait_write(b5)
        wait_idx(b5)
        issue_tgather(b5)

      @pl.when(i + NBUF < NCHUNK)
      def _():
        issue_idx(i + NBUF, b)
    return carry

  lax.fori_loop(0, NCHUNK // NBUF, loop_body, 0)

  for b in range(NBUF):
    wait_write(b)


_mesh = plsc.VectorSubcoreMesh(core_axis_name="c", subcore_axis_name="s")

_sc_embed = pl.kernel(
    _body,
    out_type=jax.ShapeDtypeStruct((N, EMBED), jnp.float32),
    mesh=_mesh,
    scratch_types=(
        [pltpu.VMEM((C,), jnp.int32) for _ in range(2 * NBUF)]
        + [pltpu.VMEM((C, EMBED), jnp.float32) for _ in range(NBUF)]
        + [pltpu.SemaphoreType.DMA for _ in range(4 * NBUF)]
        + [pltpu.VMEM_SHARED((1000, EMBED), jnp.float32)]
    ),
)


@jax.jit
def kernel(token_ids, node_ids, token_table, node_table):
  return _sc_embed(token_ids.astype(jnp.int32), node_ids.astype(jnp.int32),
                   token_table, node_table)
